# baseline (speedup 1.0000x reference)
import jax
import jax.numpy as jnp
import numpy as np
from jax.experimental import pallas as pl
from jax.experimental.pallas import tpu as pltpu

_WIDTH = 64
_K = 8192
_TEMP = 0.5
_TN = 448


def _dot(a, b):
    return jax.lax.dot_general(a, b, (((1,), (0,)), ((), ())),
                               preferred_element_type=jnp.float32)


def _quant_block(scale_ref, z_ref, caug_ref, csqr_ref, zq_ref, pt_ref, lse_ref):
    @pl.when(pl.program_id(0) == 0)
    def _init():
        pt_ref[...] = jnp.zeros((1, 1), jnp.float32)
        lse_ref[...] = jnp.zeros((1, 1), jnp.float32)

    z = z_ref[...]
    caug = caug_ref[...]
    csqr = csqr_ref[...]
    scale = scale_ref[0]

    zsq = jnp.sum(z * z, axis=1, keepdims=True)
    s = jax.lax.dot_general(z, caug[:, :_WIDTH], (((1,), (1,)), ((), ())),
                            preferred_element_type=jnp.float32)
    g = csqr - 2.0 * s
    mg = jnp.min(g, axis=1, keepdims=True)
    e = jnp.exp((g - mg) * scale)
    r = _dot(e, caug)
    den = r[:, _WIDTH + 1:_WIDTH + 2]
    ecsq = r[:, _WIDTH:_WIDTH + 1]
    zq = r[:, :_WIDTH] / den
    zq_ref[...] = zq
    zdotzq = jnp.sum(z * zq, axis=1, keepdims=True)
    pt_row = scale * (zsq + ecsq / den - 2.0 * zdotzq)
    m = scale * (zsq + mg)
    lse_row = jnp.log(den) + m
    pt_ref[...] += jnp.sum(pt_row).reshape(1, 1)
    lse_ref[...] += jnp.sum(lse_row).reshape(1, 1)


def _quantize(zf, codebook, var):
    n = zf.shape[0]
    k = codebook.shape[0]
    scale = (-0.5 / (var * _TEMP)).reshape(1).astype(jnp.float32)
    csq = jnp.sum(codebook * codebook, axis=1)
    caug = jnp.concatenate(
        [codebook, csq[:, None], jnp.ones((k, 1), jnp.float32),
         jnp.zeros((k, 128 - _WIDTH - 2), jnp.float32)], axis=1)
    csqr = csq[None, :]
    zq, pt, lse = pl.pallas_call(
        _quant_block,
        grid=(n // _TN,),
        in_specs=[
            pl.BlockSpec(memory_space=pltpu.SMEM),
            pl.BlockSpec((_TN, _WIDTH), lambda i: (i, 0)),
            pl.BlockSpec((_K, 128), lambda i: (0, 0)),
            pl.BlockSpec((1, _K), lambda i: (0, 0)),
        ],
        out_specs=[
            pl.BlockSpec((_TN, _WIDTH), lambda i: (i, 0)),
            pl.BlockSpec((1, 1), lambda i: (0, 0)),
            pl.BlockSpec((1, 1), lambda i: (0, 0)),
        ],
        out_shape=[
            jax.ShapeDtypeStruct((n, _WIDTH), jnp.float32),
            jax.ShapeDtypeStruct((1, 1), jnp.float32),
            jax.ShapeDtypeStruct((1, 1), jnp.float32),
        ],
    )(scale, zf, caug, csqr)
    return zq, pt[0, 0], lse[0, 0]


_NHWC = ('NHWC', 'HWIO', 'NHWC')


def _conv_s2(x, w, b):
    y = jax.lax.conv_general_dilated(x, w.transpose(2, 3, 1, 0), (2, 2),
                                     ((1, 1), (1, 1)), dimension_numbers=_NHWC)
    return y + b[None, None, None, :]


def _tconv_s2(x, w, b):
    wf = w[:, :, ::-1, ::-1].transpose(2, 3, 1, 0)
    y = jax.lax.conv_general_dilated(x.astype(jnp.bfloat16),
                                     wf.astype(jnp.bfloat16), (1, 1),
                                     ((2, 2), (2, 2)), lhs_dilation=(2, 2),
                                     dimension_numbers=_NHWC,
                                     preferred_element_type=jnp.float32)
    return y + b[None, None, None, :]


def kernel(x, enc_w1, enc_b1, enc_w2, enc_b2, enc_w3, enc_b3,
           dec_w1, dec_b1, dec_w2, dec_b2, dec_w3, dec_b3, codebook, log_var):
    bsz = x.shape[0]
    xh = x.transpose(0, 2, 3, 1)
    h = jax.nn.relu(_conv_s2(xh, enc_w1, enc_b1))
    h = jax.nn.relu(_conv_s2(h, enc_w2, enc_b2))
    z = _conv_s2(h, enc_w3, enc_b3)
    zf = z.reshape(bsz * 28 * 28, _WIDTH)

    var = jnp.exp(log_var)
    zq, pt_sum, lse_sum = _quantize(zf, codebook, var)
    n = zf.shape[0]
    mean_pt = pt_sum / n
    mean_lse = lse_sum / n
    loss_latent = (1.0 - _TEMP) * mean_pt - mean_lse + np.float32(np.log(_K))

    zq4 = zq.reshape(bsz, 28, 28, _WIDTH)
    h = jax.nn.relu(_tconv_s2(zq4, dec_w1, dec_b1))
    h = jax.nn.relu(_tconv_s2(h, dec_w2, dec_b2))
    xr = _tconv_s2(h, dec_w3, dec_b3)
    x_rec = xr.transpose(0, 3, 1, 2)

    dim_x = float(np.prod(x_rec.shape[1:]))
    se = jnp.sum((x_rec - x) ** 2) / bsz
    loss_rec = dim_x * jnp.log(se) / 2.0
    rmse = jnp.sqrt(se / dim_x)
    loss = loss_latent + loss_rec
    return (loss, x_rec, rmse)


# speedup vs baseline: 1.0209x; 1.0209x over previous
import jax
import jax.numpy as jnp
import numpy as np
from jax.experimental import pallas as pl
from jax.experimental.pallas import tpu as pltpu

_WIDTH = 64
_K = 8192
_TEMP = 0.5
_TN = 448


def _dot(a, b):
    return jax.lax.dot_general(a, b, (((1,), (0,)), ((), ())),
                               preferred_element_type=jnp.float32)


def _quant_block(scale_ref, z_ref, caug_ref, csqr_ref, zq_ref, pt_ref, lse_ref):
    @pl.when(pl.program_id(0) == 0)
    def _init():
        pt_ref[...] = jnp.zeros((1, 1), jnp.float32)
        lse_ref[...] = jnp.zeros((1, 1), jnp.float32)

    z = z_ref[...]
    caug = caug_ref[...]
    csqr = csqr_ref[...]
    scale = scale_ref[0]

    zsq = jnp.sum(z * z, axis=1, keepdims=True)
    s = jax.lax.dot_general(z, caug[:, :_WIDTH], (((1,), (1,)), ((), ())),
                            preferred_element_type=jnp.float32)
    g = csqr - 2.0 * s
    mg = jnp.min(g, axis=1, keepdims=True)
    e = jnp.exp((g - mg) * scale)
    r = _dot(e, caug)
    den = r[:, _WIDTH + 1:_WIDTH + 2]
    ecsq = r[:, _WIDTH:_WIDTH + 1]
    zq = r[:, :_WIDTH] / den
    zq_ref[...] = zq
    zdotzq = jnp.sum(z * zq, axis=1, keepdims=True)
    pt_row = scale * (zsq + ecsq / den - 2.0 * zdotzq)
    m = scale * (zsq + mg)
    lse_row = jnp.log(den) + m
    pt_ref[...] += jnp.sum(pt_row).reshape(1, 1)
    lse_ref[...] += jnp.sum(lse_row).reshape(1, 1)


def _quantize(zf, codebook, var):
    n = zf.shape[0]
    k = codebook.shape[0]
    scale = (-0.5 / (var * _TEMP)).reshape(1).astype(jnp.float32)
    csq = jnp.sum(codebook * codebook, axis=1)
    caug = jnp.concatenate(
        [codebook, csq[:, None], jnp.ones((k, 1), jnp.float32),
         jnp.zeros((k, 128 - _WIDTH - 2), jnp.float32)], axis=1)
    csqr = csq[None, :]
    zq, pt, lse = pl.pallas_call(
        _quant_block,
        grid=(n // _TN,),
        in_specs=[
            pl.BlockSpec(memory_space=pltpu.SMEM),
            pl.BlockSpec((_TN, _WIDTH), lambda i: (i, 0)),
            pl.BlockSpec((_K, 128), lambda i: (0, 0)),
            pl.BlockSpec((1, _K), lambda i: (0, 0)),
        ],
        out_specs=[
            pl.BlockSpec((_TN, _WIDTH), lambda i: (i, 0)),
            pl.BlockSpec((1, 1), lambda i: (0, 0)),
            pl.BlockSpec((1, 1), lambda i: (0, 0)),
        ],
        out_shape=[
            jax.ShapeDtypeStruct((n, _WIDTH), jnp.float32),
            jax.ShapeDtypeStruct((1, 1), jnp.float32),
            jax.ShapeDtypeStruct((1, 1), jnp.float32),
        ],
    )(scale, zf, caug, csqr)
    return zq, pt[0, 0], lse[0, 0]


_NHWC = ('NHWC', 'HWIO', 'NHWC')


def _conv_s2(x, w, b):
    y = jax.lax.conv_general_dilated(x, w.transpose(2, 3, 1, 0), (2, 2),
                                     ((1, 1), (1, 1)), dimension_numbers=_NHWC,
                                     precision=jax.lax.Precision.DEFAULT)
    return y + b[None, None, None, :]


def _tconv_s2(x, w, b):
    wf = w[:, :, ::-1, ::-1].transpose(2, 3, 1, 0)
    y = jax.lax.conv_general_dilated(x, wf, (1, 1), ((2, 2), (2, 2)),
                                     lhs_dilation=(2, 2),
                                     dimension_numbers=_NHWC,
                                     precision=jax.lax.Precision.DEFAULT)
    return y + b[None, None, None, :]


def kernel(x, enc_w1, enc_b1, enc_w2, enc_b2, enc_w3, enc_b3,
           dec_w1, dec_b1, dec_w2, dec_b2, dec_w3, dec_b3, codebook, log_var):
    bsz = x.shape[0]
    xh = x.transpose(0, 2, 3, 1)
    h = jax.nn.relu(_conv_s2(xh, enc_w1, enc_b1))
    h = jax.nn.relu(_conv_s2(h, enc_w2, enc_b2))
    z = _conv_s2(h, enc_w3, enc_b3)
    zf = z.reshape(bsz * 28 * 28, _WIDTH)

    var = jnp.exp(log_var)
    zq, pt_sum, lse_sum = _quantize(zf, codebook, var)
    n = zf.shape[0]
    mean_pt = pt_sum / n
    mean_lse = lse_sum / n
    loss_latent = (1.0 - _TEMP) * mean_pt - mean_lse + np.float32(np.log(_K))

    zq4 = zq.reshape(bsz, 28, 28, _WIDTH)
    h = jax.nn.relu(_tconv_s2(zq4, dec_w1, dec_b1))
    h = jax.nn.relu(_tconv_s2(h, dec_w2, dec_b2))
    xr = _tconv_s2(h, dec_w3, dec_b3)
    x_rec = xr.transpose(0, 3, 1, 2)

    dim_x = float(np.prod(x_rec.shape[1:]))
    se = jnp.sum((x_rec - x) ** 2) / bsz
    loss_rec = dim_x * jnp.log(se) / 2.0
    rmse = jnp.sqrt(se / dim_x)
    loss = loss_latent + loss_rec
    return (loss, x_rec, rmse)
